# baseline (speedup 1.0000x reference)
import functools

import jax
import jax.numpy as jnp
from jax import lax
from jax.experimental import pallas as pl
from jax.experimental.pallas import tpu as pltpu
from jax.experimental.pallas import tpu_sc as plsc

NAME_NUM = 1000000
SUBJECT_NUM = 128
MAX_LEN = 200
FACTOR_NUM = 64
BATCH = 16384
_SQ = 128

_INFO = plsc.get_sparse_core_info()
_NC = _INFO.num_cores
_NS = _INFO.num_subcores
_NW = _NC * _NS
_BPW = BATCH // _NW
_G = 16
_NG = _BPW // _G


def _k0_body(subj_ref, sw_ref, w_ref, b_ref, out_ref):
    pt = lax.dot_general(
        sw_ref[...], w_ref[...],
        dimension_numbers=(((1,), (1,)), ((), ())),
        preferred_element_type=jnp.float32,
    ) + b_ref[...]
    iota = lax.broadcasted_iota(jnp.int32, (MAX_LEN, BATCH), 0)
    oh = (subj_ref[...] == iota).astype(jnp.float32)
    out_ref[...] = lax.dot_general(
        pt, oh,
        dimension_numbers=(((0,), (0,)), ((), ())),
        preferred_element_type=jnp.float32,
    )


_k0_call = pl.pallas_call(
    _k0_body,
    out_shape=jax.ShapeDtypeStruct((FACTOR_NUM, BATCH), jnp.float32),
)


_mesh = plsc.VectorSubcoreMesh(core_axis_name="c", subcore_axis_name="s")


@functools.partial(
    pl.kernel,
    mesh=_mesh,
    compiler_params=pltpu.CompilerParams(use_tc_tiling_on_sc=False),
    out_type=jax.ShapeDtypeStruct((BATCH,), jnp.float32),
    scratch_types=[
        pltpu.VMEM((_BPW,), jnp.int32),
        pltpu.VMEM((FACTOR_NUM * _BPW,), jnp.int32),
        pltpu.VMEM((FACTOR_NUM * _BPW,), jnp.float32),
        pltpu.VMEM((FACTOR_NUM, _BPW), jnp.float32),
        pltpu.VMEM((_BPW,), jnp.float32),
        pltpu.SemaphoreType.DMA,
        pltpu.SemaphoreType.DMA,
    ],
)
def _sc_dot(name_hbm, pt_hbm, table_hbm, out_hbm,
            nidx_v, widx_v, gath_v, ptv, out_v, gsem, psem):
    wid = lax.axis_index("s") * _NC + lax.axis_index("c")
    base = wid * _BPW
    pltpu.sync_copy(name_hbm.at[pl.ds(base, _BPW)], nidx_v)
    cp_pt = pltpu.async_copy(pt_hbm.at[:, pl.ds(base, _BPW)], ptv, psem)

    def build(g, carry):
        ids = nidx_v[pl.ds(g * _G, _G)]
        for d in range(FACTOR_NUM):
            widx_v[pl.ds(d * _BPW + g * _G, _G)] = ids + d * NAME_NUM
        return carry

    lax.fori_loop(0, _NG, build, 0)
    pltpu.async_copy(table_hbm.at[widx_v], gath_v, gsem).wait()
    cp_pt.wait()

    def dot(g, carry):
        acc = jnp.zeros((16,), jnp.float32)
        for d in range(FACTOR_NUM):
            a = gath_v[pl.ds(d * _BPW + g * _G, _G)]
            p = ptv[d, pl.ds(g * _G, _G)]
            acc = acc + a * p
        out_v[pl.ds(g * _G, _G)] = acc
        return carry

    lax.fori_loop(0, _NG, dot, 0)
    pltpu.sync_copy(out_v, out_hbm.at[pl.ds(base, _BPW)])


def _p3_body(l_ref, out_ref):
    logits = l_ref[...]
    m = jnp.max(logits)
    e = jnp.exp(logits - m)
    out_ref[...] = e / jnp.sum(e)


_p3_call = pl.pallas_call(
    _p3_body,
    out_shape=jax.ShapeDtypeStruct((_SQ, _SQ), jnp.float32),
)


def kernel(subject, name, idx, embed_name_w, embed_subject_w, W, b):
    pt = _k0_call(subject.reshape(1, BATCH), embed_subject_w, W,
                  b.reshape(1, FACTOR_NUM))
    table_flat = embed_name_w.T.reshape(FACTOR_NUM * NAME_NUM)
    logits = _sc_dot(name, pt, table_flat)
    out = _p3_call(logits.reshape(_SQ, _SQ))
    return out.reshape(BATCH)
